# Initial kernel scaffold; baseline (speedup 1.0000x reference)
import jax
import jax.numpy as jnp
from jax.experimental import pallas as pl


def kernel(x, edge_index, edge_attr, W_src0, W_dst0, W_edge0, b0, W_src1, W_dst1, W_edge1, b1, W_src2, W_dst2, W_edge2, b2):
    raise NotImplementedError("write your pallas kernel here")


# speedup vs baseline: 2.6585x; 2.6585x over previous
import functools

import jax
import jax.numpy as jnp
from jax import lax
from jax.experimental import pallas as pl
from jax.experimental.pallas import tpu as pltpu
from jax.experimental.pallas import tpu_sc as plsc

N = 10000
E = 320000
D = 128
DE = 16

_NC = 2
_NS = 16
_NW = _NC * _NS
_CH = 128
_CHUNKS = 80
_EPW = _CHUNKS * _CH
_E_PAD = _NW * _EPW
_NPAD = 10112
_ZR = _NPAD // _NS

_BM = 2000


def _sc_mesh():
    return plsc.VectorSubcoreMesh(core_axis_name="c", subcore_axis_name="s",
                                  num_cores=_NC, num_subcores=_NS)


_GC = 40
_GROUPS = _CHUNKS // _GC


@functools.partial(
    pl.kernel,
    out_type=jax.ShapeDtypeStruct((_NC, _NPAD, D), jnp.float32),
    mesh=_sc_mesh(),
    scratch_types=[
        pltpu.VMEM((_GC, _CH), jnp.int32),
        pltpu.VMEM((_GC, _CH), jnp.int32),
        pltpu.VMEM((2, _CH, D), jnp.float32),
        pltpu.VMEM_SHARED((_NPAD, D), jnp.float32),
        pltpu.SemaphoreType.DMA,
        pltpu.SemaphoreType.DMA,
    ],
)
def _sc_gather_scatter(hs_hbm, src_hbm, dst_hbm, z_hbm, out_hbm,
                       src_v, dst_v, rows_v, acc, sem0, sem1):
    c = lax.axis_index("c")
    s = lax.axis_index("s")
    wid = c * _NS + s
    pltpu.sync_copy(z_hbm.at[pl.ds(s * _ZR, _ZR)], acc.at[pl.ds(s * _ZR, _ZR)])
    plsc.subcore_barrier()

    sems = (sem0, sem1)

    def group(g, carry):
        pltpu.sync_copy(src_hbm.at[wid, pl.ds(g * _GC, _GC)], src_v)
        pltpu.sync_copy(dst_hbm.at[wid, pl.ds(g * _GC, _GC)], dst_v)

        def pair(jj, c2):
            j0 = jj * 2
            for b in range(2):
                j = j0 + b
                pltpu.async_copy(hs_hbm.at[src_v.at[j]], rows_v.at[b],
                                 sems[b]).wait()
                pltpu.sync_copy(rows_v.at[b], acc.at[dst_v.at[j]], add=True)
            return c2

        lax.fori_loop(0, _GC // 2, pair, 0)
        return carry

    lax.fori_loop(0, _GROUPS, group, 0)

    plsc.subcore_barrier()
    pltpu.sync_copy(acc.at[pl.ds(s * _ZR, _ZR)],
                    out_hbm.at[c, pl.ds(s * _ZR, _ZR)])


@functools.partial(
    pl.kernel,
    out_type=jax.ShapeDtypeStruct((_NC, _NPAD, D), jnp.float32),
    mesh=_sc_mesh(),
    scratch_types=[
        pltpu.VMEM((_GC, _CH), jnp.int32),
        pltpu.VMEM((2, _CH, D), jnp.float32),
        pltpu.VMEM_SHARED((_NPAD, D), jnp.float32),
        pltpu.SemaphoreType.DMA,
        pltpu.SemaphoreType.DMA,
    ],
)
def _sc_ea_segsum(ea_hbm, dst_hbm, z_hbm, out_hbm,
                  dst_v, rows_v, acc, sem0, sem1):
    c = lax.axis_index("c")
    s = lax.axis_index("s")
    wid = c * _NS + s
    pltpu.sync_copy(z_hbm.at[pl.ds(s * _ZR, _ZR)], acc.at[pl.ds(s * _ZR, _ZR)])
    plsc.subcore_barrier()

    sems = (sem0, sem1)

    def group(g, carry):
        pltpu.sync_copy(dst_hbm.at[wid, pl.ds(g * _GC, _GC)], dst_v)

        def pair(jj, c2):
            j0 = jj * 2
            for b in range(2):
                j = j0 + b
                pltpu.async_copy(
                    ea_hbm.at[pl.ds(wid * _EPW + (g * _GC + j) * _CH, _CH)],
                    rows_v.at[b], sems[b]).wait()
                pltpu.sync_copy(rows_v.at[b], acc.at[dst_v.at[j]], add=True)
            return c2

        lax.fori_loop(0, _GC // 2, pair, 0)
        return carry

    lax.fori_loop(0, _GROUPS, group, 0)

    plsc.subcore_barrier()
    pltpu.sync_copy(acc.at[pl.ds(s * _ZR, _ZR)],
                    out_hbm.at[c, pl.ds(s * _ZR, _ZR)])


def _tc_in(x, wsT, wdT):
    def body(x_ref, ws_ref, wd_ref, hs_ref, hd_ref):
        xb = x_ref[...]
        hs_ref[...] = jnp.dot(xb, ws_ref[...], preferred_element_type=jnp.float32)
        hd_ref[...] = jnp.dot(xb, wd_ref[...], preferred_element_type=jnp.float32)

    return pl.pallas_call(
        body,
        grid=(N // _BM,),
        in_specs=[
            pl.BlockSpec((_BM, D), lambda i: (i, 0)),
            pl.BlockSpec((D, D), lambda i: (0, 0)),
            pl.BlockSpec((D, D), lambda i: (0, 0)),
        ],
        out_specs=[pl.BlockSpec((_BM, D), lambda i: (i, 0))] * 2,
        out_shape=[jax.ShapeDtypeStruct((N, D), jnp.float32)] * 2,
    )(x, wsT, wdT)


def _tc_mid(hd, agg, S, weT, b, wsT, wdT):
    def body(hd_ref, a_ref, s_ref, we_ref, b_ref, ws_ref, wd_ref,
             hs_ref, hd2_ref):
        h = hd_ref[...] + a_ref[0] + a_ref[1]
        h = h + jnp.dot(s_ref[0] + s_ref[1], we_ref[...],
                        preferred_element_type=jnp.float32)
        h = h + b_ref[...]
        h = jnp.where(h >= 0, h, 0.01 * h)
        hs_ref[...] = jnp.dot(h, ws_ref[...], preferred_element_type=jnp.float32)
        hd2_ref[...] = jnp.dot(h, wd_ref[...], preferred_element_type=jnp.float32)

    return pl.pallas_call(
        body,
        grid=(N // _BM,),
        in_specs=[
            pl.BlockSpec((_BM, D), lambda i: (i, 0)),
            pl.BlockSpec((_NC, _BM, D), lambda i: (0, i, 0)),
            pl.BlockSpec((_NC, _BM, D), lambda i: (0, i, 0)),
            pl.BlockSpec((D, D), lambda i: (0, 0)),
            pl.BlockSpec((1, D), lambda i: (0, 0)),
            pl.BlockSpec((D, D), lambda i: (0, 0)),
            pl.BlockSpec((D, D), lambda i: (0, 0)),
        ],
        out_specs=[pl.BlockSpec((_BM, D), lambda i: (i, 0))] * 2,
        out_shape=[jax.ShapeDtypeStruct((N, D), jnp.float32)] * 2,
    )(hd, agg, S, weT, b, wsT, wdT)


def _tc_out(hd, agg, S, weT, b):
    def body(hd_ref, a_ref, s_ref, we_ref, b_ref, o_ref):
        h = hd_ref[...] + a_ref[0] + a_ref[1]
        h = h + jnp.dot(s_ref[0] + s_ref[1], we_ref[...],
                        preferred_element_type=jnp.float32)
        o_ref[...] = h + b_ref[...]

    return pl.pallas_call(
        body,
        grid=(N // _BM,),
        in_specs=[
            pl.BlockSpec((_BM, D), lambda i: (i, 0)),
            pl.BlockSpec((_NC, _BM, D), lambda i: (0, i, 0)),
            pl.BlockSpec((_NC, _BM, D), lambda i: (0, i, 0)),
            pl.BlockSpec((D, D), lambda i: (0, 0)),
            pl.BlockSpec((1, D), lambda i: (0, 0)),
        ],
        out_specs=pl.BlockSpec((_BM, D), lambda i: (i, 0)),
        out_shape=jax.ShapeDtypeStruct((N, D), jnp.float32),
    )(hd, agg, S, weT, b)


def kernel(x, edge_index, edge_attr,
           W_src0, W_dst0, W_edge0, b0,
           W_src1, W_dst1, W_edge1, b1,
           W_src2, W_dst2, W_edge2, b2):
    src = edge_index[0]
    dst = edge_index[1]
    pad = _E_PAD - E
    src_p = jnp.concatenate(
        [src, jnp.zeros((pad,), jnp.int32)]).reshape(_NW, _CHUNKS, _CH)
    dst_p = jnp.concatenate(
        [dst, jnp.full((pad,), N, jnp.int32)]).reshape(_NW, _CHUNKS, _CH)
    ea_w = jnp.pad(edge_attr, ((0, pad), (0, D - DE)))
    zeros_d = jnp.zeros((_NPAD, D), jnp.float32)

    def wide(weT):
        return jnp.pad(weT, ((0, D - DE), (0, 0)))

    S = _sc_ea_segsum(ea_w, dst_p, zeros_d)

    hs, hd = _tc_in(x, W_src0.T, W_dst0.T)
    A = _sc_gather_scatter(hs, src_p, dst_p, zeros_d)
    hs, hd = _tc_mid(hd, A, S, wide(W_edge0.T), b0.reshape(1, D),
                     W_src1.T, W_dst1.T)
    A = _sc_gather_scatter(hs, src_p, dst_p, zeros_d)
    hs, hd = _tc_mid(hd, A, S, wide(W_edge1.T), b1.reshape(1, D),
                     W_src2.T, W_dst2.T)
    A = _sc_gather_scatter(hs, src_p, dst_p, zeros_d)
    return _tc_out(hd, A, S, wide(W_edge2.T), b2.reshape(1, D))


# speedup vs baseline: 2.9210x; 1.0987x over previous
import functools

import jax
import jax.numpy as jnp
from jax import lax
from jax.experimental import pallas as pl
from jax.experimental.pallas import tpu as pltpu
from jax.experimental.pallas import tpu_sc as plsc

N = 10000
E = 320000
D = 128
DE = 16

_NC = 2
_NS = 16
_NW = _NC * _NS
_CH = 128
_CHUNKS = 80
_EPW = _CHUNKS * _CH
_E_PAD = _NW * _EPW
_NPAD = 10112
_ZR = _NPAD // _NS

_BM = 2000


def _sc_mesh():
    return plsc.VectorSubcoreMesh(core_axis_name="c", subcore_axis_name="s",
                                  num_cores=_NC, num_subcores=_NS)


_GC = 16
_GROUPS = _CHUNKS // _GC


@functools.partial(
    pl.kernel,
    out_type=jax.ShapeDtypeStruct((_NC, _NPAD, D), jnp.float32),
    mesh=_sc_mesh(),
    scratch_types=[
        pltpu.VMEM((_GC, _CH), jnp.int32),
        pltpu.VMEM((_GC, _CH), jnp.int32),
        pltpu.VMEM((2, _CH, D), jnp.float32),
        pltpu.VMEM_SHARED((_NPAD, D), jnp.float32),
        pltpu.SemaphoreType.DMA,
        pltpu.SemaphoreType.DMA,
        pltpu.SemaphoreType.DMA,
        pltpu.SemaphoreType.DMA,
    ],
)
def _sc_gather_scatter(hs_hbm, src_hbm, dst_hbm, z_hbm, out_hbm,
                       src_v, dst_v, rows_v, acc, gs0, gs1, ss0, ss1):
    c = lax.axis_index("c")
    s = lax.axis_index("s")
    wid = c * _NS + s
    pltpu.sync_copy(z_hbm.at[pl.ds(s * _ZR, _ZR)], acc.at[pl.ds(s * _ZR, _ZR)])
    plsc.subcore_barrier()

    gsem = (gs0, gs1)
    ssem = (ss0, ss1)

    def group(g, carry):
        pltpu.sync_copy(src_hbm.at[wid, pl.ds(g * _GC, _GC)], src_v)
        pltpu.sync_copy(dst_hbm.at[wid, pl.ds(g * _GC, _GC)], dst_v)
        pltpu.async_copy(hs_hbm.at[src_v.at[0]], rows_v.at[0], gs0)

        def pair(jj, c2):
            j0 = jj * 2
            for b in range(2):
                j = j0 + b
                pltpu.make_async_copy(hs_hbm.at[src_v.at[j]], rows_v.at[b],
                                      gsem[b]).wait()
                pltpu.async_copy(rows_v.at[b], acc.at[dst_v.at[j]],
                                 ssem[b], add=True)

                @pl.when(j >= 1)
                def _():
                    pltpu.make_async_copy(rows_v.at[1 - b],
                                          acc.at[dst_v.at[j - 1]],
                                          ssem[1 - b]).wait()

                @pl.when(j + 1 < _GC)
                def _():
                    pltpu.async_copy(hs_hbm.at[src_v.at[j + 1]],
                                     rows_v.at[1 - b], gsem[1 - b])
            return c2

        lax.fori_loop(0, _GC // 2, pair, 0)
        pltpu.make_async_copy(rows_v.at[1], acc.at[dst_v.at[_GC - 1]],
                              ssem[1]).wait()
        return carry

    lax.fori_loop(0, _GROUPS, group, 0)

    plsc.subcore_barrier()
    pltpu.sync_copy(acc.at[pl.ds(s * _ZR, _ZR)],
                    out_hbm.at[c, pl.ds(s * _ZR, _ZR)])


@functools.partial(
    pl.kernel,
    out_type=jax.ShapeDtypeStruct((_NC, _NPAD, D), jnp.float32),
    mesh=_sc_mesh(),
    scratch_types=[
        pltpu.VMEM((_GC, _CH), jnp.int32),
        pltpu.VMEM((2, _CH, D), jnp.float32),
        pltpu.VMEM_SHARED((_NPAD, D), jnp.float32),
        pltpu.SemaphoreType.DMA,
        pltpu.SemaphoreType.DMA,
        pltpu.SemaphoreType.DMA,
        pltpu.SemaphoreType.DMA,
    ],
)
def _sc_ea_segsum(ea_hbm, dst_hbm, z_hbm, out_hbm,
                  dst_v, rows_v, acc, gs0, gs1, ss0, ss1):
    c = lax.axis_index("c")
    s = lax.axis_index("s")
    wid = c * _NS + s
    pltpu.sync_copy(z_hbm.at[pl.ds(s * _ZR, _ZR)], acc.at[pl.ds(s * _ZR, _ZR)])
    plsc.subcore_barrier()

    gsem = (gs0, gs1)
    ssem = (ss0, ss1)

    def ld(g, j):
        return ea_hbm.at[pl.ds(wid * _EPW + (g * _GC + j) * _CH, _CH)]

    def group(g, carry):
        pltpu.sync_copy(dst_hbm.at[wid, pl.ds(g * _GC, _GC)], dst_v)
        pltpu.async_copy(ld(g, 0), rows_v.at[0], gs0)

        def pair(jj, c2):
            j0 = jj * 2
            for b in range(2):
                j = j0 + b
                pltpu.make_async_copy(ld(g, j), rows_v.at[b], gsem[b]).wait()
                pltpu.async_copy(rows_v.at[b], acc.at[dst_v.at[j]],
                                 ssem[b], add=True)

                @pl.when(j >= 1)
                def _():
                    pltpu.make_async_copy(rows_v.at[1 - b],
                                          acc.at[dst_v.at[j - 1]],
                                          ssem[1 - b]).wait()

                @pl.when(j + 1 < _GC)
                def _():
                    pltpu.async_copy(ld(g, j + 1), rows_v.at[1 - b],
                                     gsem[1 - b])
            return c2

        lax.fori_loop(0, _GC // 2, pair, 0)
        pltpu.make_async_copy(rows_v.at[1], acc.at[dst_v.at[_GC - 1]],
                              ssem[1]).wait()
        return carry

    lax.fori_loop(0, _GROUPS, group, 0)

    plsc.subcore_barrier()
    pltpu.sync_copy(acc.at[pl.ds(s * _ZR, _ZR)],
                    out_hbm.at[c, pl.ds(s * _ZR, _ZR)])


def _tc_in(x, wsT, wdT):
    def body(x_ref, ws_ref, wd_ref, hs_ref, hd_ref):
        xb = x_ref[...]
        hs_ref[...] = jnp.dot(xb, ws_ref[...], preferred_element_type=jnp.float32)
        hd_ref[...] = jnp.dot(xb, wd_ref[...], preferred_element_type=jnp.float32)

    return pl.pallas_call(
        body,
        grid=(N // _BM,),
        in_specs=[
            pl.BlockSpec((_BM, D), lambda i: (i, 0)),
            pl.BlockSpec((D, D), lambda i: (0, 0)),
            pl.BlockSpec((D, D), lambda i: (0, 0)),
        ],
        out_specs=[pl.BlockSpec((_BM, D), lambda i: (i, 0))] * 2,
        out_shape=[jax.ShapeDtypeStruct((N, D), jnp.float32)] * 2,
    )(x, wsT, wdT)


def _tc_mid(hd, agg, S, weT, b, wsT, wdT):
    def body(hd_ref, a_ref, s_ref, we_ref, b_ref, ws_ref, wd_ref,
             hs_ref, hd2_ref):
        h = hd_ref[...] + a_ref[0] + a_ref[1]
        h = h + jnp.dot(s_ref[0] + s_ref[1], we_ref[...],
                        preferred_element_type=jnp.float32)
        h = h + b_ref[...]
        h = jnp.where(h >= 0, h, 0.01 * h)
        hs_ref[...] = jnp.dot(h, ws_ref[...], preferred_element_type=jnp.float32)
        hd2_ref[...] = jnp.dot(h, wd_ref[...], preferred_element_type=jnp.float32)

    return pl.pallas_call(
        body,
        grid=(N // _BM,),
        in_specs=[
            pl.BlockSpec((_BM, D), lambda i: (i, 0)),
            pl.BlockSpec((_NC, _BM, D), lambda i: (0, i, 0)),
            pl.BlockSpec((_NC, _BM, D), lambda i: (0, i, 0)),
            pl.BlockSpec((D, D), lambda i: (0, 0)),
            pl.BlockSpec((1, D), lambda i: (0, 0)),
            pl.BlockSpec((D, D), lambda i: (0, 0)),
            pl.BlockSpec((D, D), lambda i: (0, 0)),
        ],
        out_specs=[pl.BlockSpec((_BM, D), lambda i: (i, 0))] * 2,
        out_shape=[jax.ShapeDtypeStruct((N, D), jnp.float32)] * 2,
    )(hd, agg, S, weT, b, wsT, wdT)


def _tc_out(hd, agg, S, weT, b):
    def body(hd_ref, a_ref, s_ref, we_ref, b_ref, o_ref):
        h = hd_ref[...] + a_ref[0] + a_ref[1]
        h = h + jnp.dot(s_ref[0] + s_ref[1], we_ref[...],
                        preferred_element_type=jnp.float32)
        o_ref[...] = h + b_ref[...]

    return pl.pallas_call(
        body,
        grid=(N // _BM,),
        in_specs=[
            pl.BlockSpec((_BM, D), lambda i: (i, 0)),
            pl.BlockSpec((_NC, _BM, D), lambda i: (0, i, 0)),
            pl.BlockSpec((_NC, _BM, D), lambda i: (0, i, 0)),
            pl.BlockSpec((D, D), lambda i: (0, 0)),
            pl.BlockSpec((1, D), lambda i: (0, 0)),
        ],
        out_specs=pl.BlockSpec((_BM, D), lambda i: (i, 0)),
        out_shape=jax.ShapeDtypeStruct((N, D), jnp.float32),
    )(hd, agg, S, weT, b)


def kernel(x, edge_index, edge_attr,
           W_src0, W_dst0, W_edge0, b0,
           W_src1, W_dst1, W_edge1, b1,
           W_src2, W_dst2, W_edge2, b2):
    src = edge_index[0]
    dst = edge_index[1]
    pad = _E_PAD - E
    src_p = jnp.concatenate(
        [src, jnp.zeros((pad,), jnp.int32)]).reshape(_NW, _CHUNKS, _CH)
    dst_p = jnp.concatenate(
        [dst, jnp.full((pad,), N, jnp.int32)]).reshape(_NW, _CHUNKS, _CH)
    ea_w = jnp.pad(edge_attr, ((0, pad), (0, D - DE)))
    zeros_d = jnp.zeros((_NPAD, D), jnp.float32)

    def wide(weT):
        return jnp.pad(weT, ((0, D - DE), (0, 0)))

    S = _sc_ea_segsum(ea_w, dst_p, zeros_d)

    hs, hd = _tc_in(x, W_src0.T, W_dst0.T)
    A = _sc_gather_scatter(hs, src_p, dst_p, zeros_d)
    hs, hd = _tc_mid(hd, A, S, wide(W_edge0.T), b0.reshape(1, D),
                     W_src1.T, W_dst1.T)
    A = _sc_gather_scatter(hs, src_p, dst_p, zeros_d)
    hs, hd = _tc_mid(hd, A, S, wide(W_edge1.T), b1.reshape(1, D),
                     W_src2.T, W_dst2.T)
    A = _sc_gather_scatter(hs, src_p, dst_p, zeros_d)
    return _tc_out(hd, A, S, wide(W_edge2.T), b2.reshape(1, D))


# speedup vs baseline: 7.4084x; 2.5363x over previous
import functools

import jax
import jax.numpy as jnp
from jax import lax
from jax.experimental import pallas as pl
from jax.experimental.pallas import tpu as pltpu
from jax.experimental.pallas import tpu_sc as plsc

N = 10000
E = 320000
D = 128
DE = 16

_NC = 2
_NS = 16
_NW = _NC * _NS
_CH = 128
_CHUNKS = 80
_EPW = _CHUNKS * _CH
_E_PAD = _NW * _EPW
_NPAD = 10112
_ZR = _NPAD // _NS

_BM = 2000


def _sc_mesh():
    return plsc.VectorSubcoreMesh(core_axis_name="c", subcore_axis_name="s",
                                  num_cores=_NC, num_subcores=_NS)


_GC = 16
_GROUPS = _CHUNKS // _GC


@functools.partial(
    pl.kernel,
    out_type=jax.ShapeDtypeStruct((_NC, _NPAD, D), jnp.float32),
    mesh=_sc_mesh(),
    scratch_types=[
        pltpu.VMEM((_GC, _CH), jnp.int32),
        pltpu.VMEM((_GC, _CH), jnp.int32),
        pltpu.VMEM((2, _CH, D), jnp.float32),
        pltpu.VMEM_SHARED((_NPAD, D), jnp.float32),
        pltpu.SemaphoreType.DMA,
        pltpu.SemaphoreType.DMA,
        pltpu.SemaphoreType.DMA,
        pltpu.SemaphoreType.DMA,
    ],
)
def _sc_gather_scatter(hs_hbm, src_hbm, dst_hbm, z_hbm, out_hbm,
                       src_v, dst_v, rows_v, acc, gs0, gs1, ss0, ss1):
    c = lax.axis_index("c")
    s = lax.axis_index("s")
    wid = c * _NS + s
    pltpu.sync_copy(z_hbm.at[pl.ds(s * _ZR, _ZR)], acc.at[pl.ds(s * _ZR, _ZR)])
    plsc.subcore_barrier()

    gsem = (gs0, gs1)
    ssem = (ss0, ss1)

    def group(g, carry):
        pltpu.sync_copy(src_hbm.at[wid, pl.ds(g * _GC, _GC)], src_v)
        pltpu.sync_copy(dst_hbm.at[wid, pl.ds(g * _GC, _GC)], dst_v)
        pltpu.async_copy(hs_hbm.at[src_v.at[0]], rows_v.at[0], gs0)

        def pair(jj, c2):
            j0 = jj * 2
            for b in range(2):
                j = j0 + b
                pltpu.make_async_copy(hs_hbm.at[src_v.at[j]], rows_v.at[b],
                                      gsem[b]).wait()
                pltpu.async_copy(rows_v.at[b], acc.at[dst_v.at[j]],
                                 ssem[b], add=True)

                @pl.when(j >= 1)
                def _():
                    pltpu.make_async_copy(rows_v.at[1 - b],
                                          acc.at[dst_v.at[j - 1]],
                                          ssem[1 - b]).wait()

                @pl.when(j + 1 < _GC)
                def _():
                    pltpu.async_copy(hs_hbm.at[src_v.at[j + 1]],
                                     rows_v.at[1 - b], gsem[1 - b])
            return c2

        lax.fori_loop(0, _GC // 2, pair, 0)
        pltpu.make_async_copy(rows_v.at[1], acc.at[dst_v.at[_GC - 1]],
                              ssem[1]).wait()
        return carry

    lax.fori_loop(0, _GROUPS, group, 0)

    plsc.subcore_barrier()
    pltpu.sync_copy(acc.at[pl.ds(s * _ZR, _ZR)],
                    out_hbm.at[c, pl.ds(s * _ZR, _ZR)])


@functools.partial(
    pl.kernel,
    out_type=jax.ShapeDtypeStruct((_NC, _NPAD, D), jnp.float32),
    mesh=_sc_mesh(),
    scratch_types=[
        pltpu.VMEM((_GC, _CH), jnp.int32),
        pltpu.VMEM((2, _CH, D), jnp.float32),
        pltpu.VMEM_SHARED((_NPAD, D), jnp.float32),
        pltpu.SemaphoreType.DMA,
        pltpu.SemaphoreType.DMA,
        pltpu.SemaphoreType.DMA,
        pltpu.SemaphoreType.DMA,
    ],
)
def _sc_ea_segsum(ea_hbm, dst_hbm, z_hbm, out_hbm,
                  dst_v, rows_v, acc, gs0, gs1, ss0, ss1):
    c = lax.axis_index("c")
    s = lax.axis_index("s")
    wid = c * _NS + s
    pltpu.sync_copy(z_hbm.at[pl.ds(s * _ZR, _ZR)], acc.at[pl.ds(s * _ZR, _ZR)])
    plsc.subcore_barrier()

    gsem = (gs0, gs1)
    ssem = (ss0, ss1)

    def ld(g, j):
        return ea_hbm.at[pl.ds(wid * _EPW + (g * _GC + j) * _CH, _CH)]

    def group(g, carry):
        pltpu.sync_copy(dst_hbm.at[wid, pl.ds(g * _GC, _GC)], dst_v)
        pltpu.async_copy(ld(g, 0), rows_v.at[0], gs0)

        def pair(jj, c2):
            j0 = jj * 2
            for b in range(2):
                j = j0 + b
                pltpu.make_async_copy(ld(g, j), rows_v.at[b], gsem[b]).wait()
                pltpu.async_copy(rows_v.at[b], acc.at[dst_v.at[j]],
                                 ssem[b], add=True)

                @pl.when(j >= 1)
                def _():
                    pltpu.make_async_copy(rows_v.at[1 - b],
                                          acc.at[dst_v.at[j - 1]],
                                          ssem[1 - b]).wait()

                @pl.when(j + 1 < _GC)
                def _():
                    pltpu.async_copy(ld(g, j + 1), rows_v.at[1 - b],
                                     gsem[1 - b])
            return c2

        lax.fori_loop(0, _GC // 2, pair, 0)
        pltpu.make_async_copy(rows_v.at[1], acc.at[dst_v.at[_GC - 1]],
                              ssem[1]).wait()
        return carry

    lax.fori_loop(0, _GROUPS, group, 0)

    plsc.subcore_barrier()
    pltpu.sync_copy(acc.at[pl.ds(s * _ZR, _ZR)],
                    out_hbm.at[c, pl.ds(s * _ZR, _ZR)])


def _tc_in(x, wsT, wdT):
    def body(x_ref, ws_ref, wd_ref, hs_ref, hd_ref):
        xb = x_ref[...]
        hs_ref[...] = jnp.dot(xb, ws_ref[...], preferred_element_type=jnp.float32)
        hd_ref[...] = jnp.dot(xb, wd_ref[...], preferred_element_type=jnp.float32)

    return pl.pallas_call(
        body,
        grid=(N // _BM,),
        in_specs=[
            pl.BlockSpec((_BM, D), lambda i: (i, 0)),
            pl.BlockSpec((D, D), lambda i: (0, 0)),
            pl.BlockSpec((D, D), lambda i: (0, 0)),
        ],
        out_specs=[pl.BlockSpec((_BM, D), lambda i: (i, 0))] * 2,
        out_shape=[jax.ShapeDtypeStruct((N, D), jnp.float32)] * 2,
    )(x, wsT, wdT)


def _tc_mid(hd, agg, S, weT, b, wsT, wdT):
    def body(hd_ref, a_ref, s_ref, we_ref, b_ref, ws_ref, wd_ref,
             hs_ref, hd2_ref):
        h = hd_ref[...] + a_ref[0] + a_ref[1]
        h = h + jnp.dot(s_ref[0] + s_ref[1], we_ref[...],
                        preferred_element_type=jnp.float32)
        h = h + b_ref[...]
        h = jnp.where(h >= 0, h, 0.01 * h)
        hs_ref[...] = jnp.dot(h, ws_ref[...], preferred_element_type=jnp.float32)
        hd2_ref[...] = jnp.dot(h, wd_ref[...], preferred_element_type=jnp.float32)

    return pl.pallas_call(
        body,
        grid=(N // _BM,),
        in_specs=[
            pl.BlockSpec((_BM, D), lambda i: (i, 0)),
            pl.BlockSpec((_NC, _BM, D), lambda i: (0, i, 0)),
            pl.BlockSpec((_NC, _BM, D), lambda i: (0, i, 0)),
            pl.BlockSpec((D, D), lambda i: (0, 0)),
            pl.BlockSpec((1, D), lambda i: (0, 0)),
            pl.BlockSpec((D, D), lambda i: (0, 0)),
            pl.BlockSpec((D, D), lambda i: (0, 0)),
        ],
        out_specs=[pl.BlockSpec((_BM, D), lambda i: (i, 0))] * 2,
        out_shape=[jax.ShapeDtypeStruct((N, D), jnp.float32)] * 2,
    )(hd, agg, S, weT, b, wsT, wdT)


def _tc_out(hd, agg, S, weT, b):
    def body(hd_ref, a_ref, s_ref, we_ref, b_ref, o_ref):
        h = hd_ref[...] + a_ref[0] + a_ref[1]
        h = h + jnp.dot(s_ref[0] + s_ref[1], we_ref[...],
                        preferred_element_type=jnp.float32)
        o_ref[...] = h + b_ref[...]

    return pl.pallas_call(
        body,
        grid=(N // _BM,),
        in_specs=[
            pl.BlockSpec((_BM, D), lambda i: (i, 0)),
            pl.BlockSpec((_NC, _BM, D), lambda i: (0, i, 0)),
            pl.BlockSpec((_NC, _BM, D), lambda i: (0, i, 0)),
            pl.BlockSpec((D, D), lambda i: (0, 0)),
            pl.BlockSpec((1, D), lambda i: (0, 0)),
        ],
        out_specs=pl.BlockSpec((_BM, D), lambda i: (i, 0)),
        out_shape=jax.ShapeDtypeStruct((N, D), jnp.float32),
    )(hd, agg, S, weT, b)


def kernel(x, edge_index, edge_attr,
           W_src0, W_dst0, W_edge0, b0,
           W_src1, W_dst1, W_edge1, b1,
           W_src2, W_dst2, W_edge2, b2):
    src = edge_index[0]
    dst = edge_index[1]
    pad = _E_PAD - E
    pad_src = (jnp.arange(pad, dtype=jnp.int32) * 79) % N
    src_p = jnp.concatenate([src, pad_src]).reshape(_NW, _CHUNKS, _CH)
    dst_p = jnp.concatenate(
        [dst, jnp.full((pad,), N, jnp.int32)]).reshape(_NW, _CHUNKS, _CH)
    ea_w = jnp.pad(edge_attr, ((0, pad), (0, D - DE)))
    zeros_d = jnp.zeros((_NPAD, D), jnp.float32)

    def wide(weT):
        return jnp.pad(weT, ((0, D - DE), (0, 0)))

    S = _sc_ea_segsum(ea_w, dst_p, zeros_d)

    hs, hd = _tc_in(x, W_src0.T, W_dst0.T)
    A = _sc_gather_scatter(hs, src_p, dst_p, zeros_d)
    hs, hd = _tc_mid(hd, A, S, wide(W_edge0.T), b0.reshape(1, D),
                     W_src1.T, W_dst1.T)
    A = _sc_gather_scatter(hs, src_p, dst_p, zeros_d)
    hs, hd = _tc_mid(hd, A, S, wide(W_edge1.T), b1.reshape(1, D),
                     W_src2.T, W_dst2.T)
    A = _sc_gather_scatter(hs, src_p, dst_p, zeros_d)
    return _tc_out(hd, A, S, wide(W_edge2.T), b2.reshape(1, D))


# speedup vs baseline: 7.6410x; 1.0314x over previous
import functools

import jax
import jax.numpy as jnp
from jax import lax
from jax.experimental import pallas as pl
from jax.experimental.pallas import tpu as pltpu
from jax.experimental.pallas import tpu_sc as plsc

N = 10000
E = 320000
D = 128
DE = 16

_NC = 2
_NS = 16
_NW = _NC * _NS
_CH = 128
_CHUNKS = 80
_EPW = _CHUNKS * _CH
_E_PAD = _NW * _EPW
_NPAD = 10112
_ZR = _NPAD // _NS

_BM = 2000


def _sc_mesh():
    return plsc.VectorSubcoreMesh(core_axis_name="c", subcore_axis_name="s",
                                  num_cores=_NC, num_subcores=_NS)


_GC = 40
_GROUPS = _CHUNKS // _GC


@functools.partial(
    pl.kernel,
    out_type=jax.ShapeDtypeStruct((_NC, _NPAD, D), jnp.float32),
    mesh=_sc_mesh(),
    scratch_types=[
        pltpu.VMEM((_GC, _CH), jnp.int32),
        pltpu.VMEM((_GC, _CH), jnp.int32),
        pltpu.VMEM((2, _CH, D), jnp.float32),
        pltpu.VMEM_SHARED((_NPAD, D), jnp.float32),
        pltpu.SemaphoreType.DMA,
        pltpu.SemaphoreType.DMA,
        pltpu.SemaphoreType.DMA,
        pltpu.SemaphoreType.DMA,
    ],
)
def _sc_gather_scatter(hs_hbm, src_hbm, dst_hbm, z_hbm, out_hbm,
                       src_v, dst_v, rows_v, acc, gs0, gs1, ss0, ss1):
    c = lax.axis_index("c")
    s = lax.axis_index("s")
    wid = c * _NS + s
    pltpu.sync_copy(z_hbm.at[pl.ds(s * _ZR, _ZR)], acc.at[pl.ds(s * _ZR, _ZR)])
    plsc.subcore_barrier()

    gsem = (gs0, gs1)
    ssem = (ss0, ss1)

    def group(g, carry):
        pltpu.sync_copy(src_hbm.at[wid, pl.ds(g * _GC, _GC)], src_v)
        pltpu.sync_copy(dst_hbm.at[wid, pl.ds(g * _GC, _GC)], dst_v)
        pltpu.async_copy(hs_hbm.at[src_v.at[0]], rows_v.at[0], gs0)

        def pair(jj, c2):
            j0 = jj * 2
            for b in range(2):
                j = j0 + b
                pltpu.make_async_copy(hs_hbm.at[src_v.at[j]], rows_v.at[b],
                                      gsem[b]).wait()
                pltpu.async_copy(rows_v.at[b], acc.at[dst_v.at[j]],
                                 ssem[b], add=True)

                @pl.when(j >= 1)
                def _():
                    pltpu.make_async_copy(rows_v.at[1 - b],
                                          acc.at[dst_v.at[j - 1]],
                                          ssem[1 - b]).wait()

                @pl.when(j + 1 < _GC)
                def _():
                    pltpu.async_copy(hs_hbm.at[src_v.at[j + 1]],
                                     rows_v.at[1 - b], gsem[1 - b])
            return c2

        lax.fori_loop(0, _GC // 2, pair, 0)
        pltpu.make_async_copy(rows_v.at[1], acc.at[dst_v.at[_GC - 1]],
                              ssem[1]).wait()
        return carry

    lax.fori_loop(0, _GROUPS, group, 0)

    plsc.subcore_barrier()
    pltpu.sync_copy(acc.at[pl.ds(s * _ZR, _ZR)],
                    out_hbm.at[c, pl.ds(s * _ZR, _ZR)])


@functools.partial(
    pl.kernel,
    out_type=jax.ShapeDtypeStruct((_NC, _NPAD, D), jnp.float32),
    mesh=_sc_mesh(),
    scratch_types=[
        pltpu.VMEM((_GC, _CH), jnp.int32),
        pltpu.VMEM((2, _CH, D), jnp.float32),
        pltpu.VMEM_SHARED((_NPAD, D), jnp.float32),
        pltpu.SemaphoreType.DMA,
        pltpu.SemaphoreType.DMA,
        pltpu.SemaphoreType.DMA,
        pltpu.SemaphoreType.DMA,
    ],
)
def _sc_ea_segsum(ea_hbm, dst_hbm, z_hbm, out_hbm,
                  dst_v, rows_v, acc, gs0, gs1, ss0, ss1):
    c = lax.axis_index("c")
    s = lax.axis_index("s")
    wid = c * _NS + s
    pltpu.sync_copy(z_hbm.at[pl.ds(s * _ZR, _ZR)], acc.at[pl.ds(s * _ZR, _ZR)])
    plsc.subcore_barrier()

    gsem = (gs0, gs1)
    ssem = (ss0, ss1)

    def ld(g, j):
        return ea_hbm.at[pl.ds(wid * _EPW + (g * _GC + j) * _CH, _CH)]

    def group(g, carry):
        pltpu.sync_copy(dst_hbm.at[wid, pl.ds(g * _GC, _GC)], dst_v)
        pltpu.async_copy(ld(g, 0), rows_v.at[0], gs0)

        def pair(jj, c2):
            j0 = jj * 2
            for b in range(2):
                j = j0 + b
                pltpu.make_async_copy(ld(g, j), rows_v.at[b], gsem[b]).wait()
                pltpu.async_copy(rows_v.at[b], acc.at[dst_v.at[j]],
                                 ssem[b], add=True)

                @pl.when(j >= 1)
                def _():
                    pltpu.make_async_copy(rows_v.at[1 - b],
                                          acc.at[dst_v.at[j - 1]],
                                          ssem[1 - b]).wait()

                @pl.when(j + 1 < _GC)
                def _():
                    pltpu.async_copy(ld(g, j + 1), rows_v.at[1 - b],
                                     gsem[1 - b])
            return c2

        lax.fori_loop(0, _GC // 2, pair, 0)
        pltpu.make_async_copy(rows_v.at[1], acc.at[dst_v.at[_GC - 1]],
                              ssem[1]).wait()
        return carry

    lax.fori_loop(0, _GROUPS, group, 0)

    plsc.subcore_barrier()
    pltpu.sync_copy(acc.at[pl.ds(s * _ZR, _ZR)],
                    out_hbm.at[c, pl.ds(s * _ZR, _ZR)])


def _tc_in(x, wsT, wdT):
    def body(x_ref, ws_ref, wd_ref, hs_ref, hd_ref):
        xb = x_ref[...]
        hs_ref[...] = jnp.dot(xb, ws_ref[...], preferred_element_type=jnp.float32)
        hd_ref[...] = jnp.dot(xb, wd_ref[...], preferred_element_type=jnp.float32)

    return pl.pallas_call(
        body,
        grid=(N // _BM,),
        in_specs=[
            pl.BlockSpec((_BM, D), lambda i: (i, 0)),
            pl.BlockSpec((D, D), lambda i: (0, 0)),
            pl.BlockSpec((D, D), lambda i: (0, 0)),
        ],
        out_specs=[pl.BlockSpec((_BM, D), lambda i: (i, 0))] * 2,
        out_shape=[jax.ShapeDtypeStruct((N, D), jnp.float32)] * 2,
    )(x, wsT, wdT)


def _tc_mid(hd, agg, S, weT, b, wsT, wdT):
    def body(hd_ref, a_ref, s_ref, we_ref, b_ref, ws_ref, wd_ref,
             hs_ref, hd2_ref):
        h = hd_ref[...] + a_ref[0] + a_ref[1]
        h = h + jnp.dot(s_ref[0] + s_ref[1], we_ref[...],
                        preferred_element_type=jnp.float32)
        h = h + b_ref[...]
        h = jnp.where(h >= 0, h, 0.01 * h)
        hs_ref[...] = jnp.dot(h, ws_ref[...], preferred_element_type=jnp.float32)
        hd2_ref[...] = jnp.dot(h, wd_ref[...], preferred_element_type=jnp.float32)

    return pl.pallas_call(
        body,
        grid=(N // _BM,),
        in_specs=[
            pl.BlockSpec((_BM, D), lambda i: (i, 0)),
            pl.BlockSpec((_NC, _BM, D), lambda i: (0, i, 0)),
            pl.BlockSpec((_NC, _BM, D), lambda i: (0, i, 0)),
            pl.BlockSpec((D, D), lambda i: (0, 0)),
            pl.BlockSpec((1, D), lambda i: (0, 0)),
            pl.BlockSpec((D, D), lambda i: (0, 0)),
            pl.BlockSpec((D, D), lambda i: (0, 0)),
        ],
        out_specs=[pl.BlockSpec((_BM, D), lambda i: (i, 0))] * 2,
        out_shape=[jax.ShapeDtypeStruct((N, D), jnp.float32)] * 2,
    )(hd, agg, S, weT, b, wsT, wdT)


def _tc_out(hd, agg, S, weT, b):
    def body(hd_ref, a_ref, s_ref, we_ref, b_ref, o_ref):
        h = hd_ref[...] + a_ref[0] + a_ref[1]
        h = h + jnp.dot(s_ref[0] + s_ref[1], we_ref[...],
                        preferred_element_type=jnp.float32)
        o_ref[...] = h + b_ref[...]

    return pl.pallas_call(
        body,
        grid=(N // _BM,),
        in_specs=[
            pl.BlockSpec((_BM, D), lambda i: (i, 0)),
            pl.BlockSpec((_NC, _BM, D), lambda i: (0, i, 0)),
            pl.BlockSpec((_NC, _BM, D), lambda i: (0, i, 0)),
            pl.BlockSpec((D, D), lambda i: (0, 0)),
            pl.BlockSpec((1, D), lambda i: (0, 0)),
        ],
        out_specs=pl.BlockSpec((_BM, D), lambda i: (i, 0)),
        out_shape=jax.ShapeDtypeStruct((N, D), jnp.float32),
    )(hd, agg, S, weT, b)


def kernel(x, edge_index, edge_attr,
           W_src0, W_dst0, W_edge0, b0,
           W_src1, W_dst1, W_edge1, b1,
           W_src2, W_dst2, W_edge2, b2):
    src = edge_index[0]
    dst = edge_index[1]
    pad = _E_PAD - E
    pad_src = (jnp.arange(pad, dtype=jnp.int32) * 79) % N
    src_p = jnp.concatenate([src, pad_src]).reshape(_NW, _CHUNKS, _CH)
    dst_p = jnp.concatenate(
        [dst, jnp.full((pad,), N, jnp.int32)]).reshape(_NW, _CHUNKS, _CH)
    ea_w = jnp.pad(edge_attr, ((0, pad), (0, D - DE)))
    zeros_d = jnp.zeros((_NPAD, D), jnp.float32)

    def wide(weT):
        return jnp.pad(weT, ((0, D - DE), (0, 0)))

    hs, hd = _tc_in(x, W_src0.T, W_dst0.T)
    A = _sc_gather_scatter(hs, src_p, dst_p, zeros_d)
    S = _sc_ea_segsum(ea_w, dst_p, zeros_d)
    hs, hd = _tc_mid(hd, A, S, wide(W_edge0.T), b0.reshape(1, D),
                     W_src1.T, W_dst1.T)
    A = _sc_gather_scatter(hs, src_p, dst_p, zeros_d)
    hs, hd = _tc_mid(hd, A, S, wide(W_edge1.T), b1.reshape(1, D),
                     W_src2.T, W_dst2.T)
    A = _sc_gather_scatter(hs, src_p, dst_p, zeros_d)
    return _tc_out(hd, A, S, wide(W_edge2.T), b2.reshape(1, D))
